# baseline (speedup 1.0000x reference)
import functools

import jax
import jax.numpy as jnp
from jax import lax
from jax.experimental import pallas as pl
from jax.experimental.pallas import tpu as pltpu
from jax.experimental.pallas import tpu_sc as plsc

N_EMBED = 1000
D_MODEL = 128
LENGTH = 200
BATCH = 4096
SCALE = float(D_MODEL) ** 0.5

NUM_CORES = 2
NUM_SUBCORES = 16
NUM_WORKERS = NUM_CORES * NUM_SUBCORES
ROWS_PER_WORKER = BATCH // NUM_WORKERS
LANES = 16
GROUPS = ROWS_PER_WORKER // LANES
D_CHUNKS = D_MODEL // LANES


ROWS_PER_TILE = 63


def _embed_body(
    xt_hbm, band_hbm, pos_hbm, out_hbm,
    xg, idxt, posv, buf0, buf1, table_sh, sg0, sg1, so0, so1,
):
    c = lax.axis_index("c")
    s = lax.axis_index("s")
    wid = s * NUM_CORES + c
    base = wid * ROWS_PER_WORKER

    @pl.when(s == 0)
    def _():
        pltpu.async_copy(band_hbm, table_sh, sg0)

    pltpu.sync_copy(pos_hbm, posv)

    def group_body(g, carry):
        gbase = base + g * LANES
        pltpu.sync_copy(xt_hbm.at[:, pl.ds(gbase, LANES)], xg)

        def max_body(l, m):
            return jnp.maximum(m, xg[l])

        m = lax.fori_loop(
            0, LENGTH, max_body, jnp.full((LANES,), -jnp.inf, jnp.float32)
        )

        def quant_body(l, carry2):
            v = xg[l] / m * jnp.float32(N_EMBED - 1)
            v = jnp.where(v < 0.0, 0.0, v)
            idxt[l, pl.ds(g * LANES, LANES)] = v.astype(jnp.int32)
            return carry2

        lax.fori_loop(0, LENGTH, quant_body, 0)
        return carry

    lax.fori_loop(0, GROUPS, group_body, 0)

    @pl.when(s == 0)
    def _():
        pltpu.make_async_copy(band_hbm, table_sh, sg0).wait()

    plsc.subcore_barrier()

    out_slice = out_hbm.at[pl.ds(base, ROWS_PER_WORKER)]

    def compute(buf, l):
        pv = [posv[l, pl.ds(j * LANES, LANES)] for j in range(D_CHUNKS)]

        @plsc.parallel_loop(0, ROWS_PER_WORKER, unroll=8)
        def add_body(t):
            for j in range(D_CHUNKS):
                sl = pl.ds(j * LANES, LANES)
                buf[t, sl] = buf[t, sl] * SCALE + pv[j]

    def gather_issue(l, buf, sem):
        pltpu.async_copy(table_sh.at[idxt.at[l]], buf, sem)

    def gather_wait(l, buf, sem):
        pltpu.make_async_copy(table_sh.at[idxt.at[l]], buf, sem).wait()

    def out_issue(l, buf, sem):
        pltpu.async_copy(buf, out_slice.at[:, l], sem)

    def out_wait(l, buf, sem):
        pltpu.make_async_copy(buf, out_slice.at[:, l], sem).wait()

    gather_issue(0, buf0, sg0)

    def pair_body(k, carry):
        l0 = 2 * k
        l1 = l0 + 1
        gather_wait(l0, buf0, sg0)

        @pl.when(k >= 1)
        def _():
            out_wait(l1, buf1, so1)

        gather_issue(l1, buf1, sg1)
        compute(buf0, l0)
        out_issue(l0, buf0, so0)
        gather_wait(l1, buf1, sg1)
        compute(buf1, l1)
        out_issue(l1, buf1, so1)
        out_wait(l0, buf0, so0)

        @pl.when(k < LENGTH // 2 - 1)
        def _():
            gather_issue(l0 + 2, buf0, sg0)

        return carry

    lax.fori_loop(0, LENGTH // 2, pair_body, 0)
    out_wait(LENGTH - 1, buf1, so1)


def kernel(x, embed_band, embed_pos):
    xt = x.reshape(BATCH, LENGTH).T
    mesh = plsc.VectorSubcoreMesh(core_axis_name="c", subcore_axis_name="s")
    k = functools.partial(
        pl.kernel,
        mesh=mesh,
        compiler_params=pltpu.CompilerParams(
            use_tc_tiling_on_sc=False, needs_layout_passes=False
        ),
        out_type=jax.ShapeDtypeStruct((BATCH, LENGTH, D_MODEL), jnp.float32),
        scratch_types=[
            pltpu.VMEM((LENGTH, LANES), jnp.float32),
            pltpu.VMEM((LENGTH, ROWS_PER_WORKER), jnp.int32),
            pltpu.VMEM((LENGTH, D_MODEL), jnp.float32),
            pltpu.VMEM((ROWS_PER_WORKER, D_MODEL), jnp.float32),
            pltpu.VMEM((ROWS_PER_WORKER, D_MODEL), jnp.float32),
            pltpu.VMEM_SHARED((N_EMBED, D_MODEL), jnp.float32),
            pltpu.SemaphoreType.DMA,
            pltpu.SemaphoreType.DMA,
            pltpu.SemaphoreType.DMA,
            pltpu.SemaphoreType.DMA,
        ],
    )(_embed_body)
    return k(xt, embed_band, embed_pos)
